# baseline (speedup 1.0000x reference)
import jax
import jax.numpy as jnp
from jax import lax
from jax.experimental import pallas as pl
from jax.experimental.pallas import tpu as pltpu
import jax.experimental.pallas.tpu_sc as plsc

N = 10000
E = 320000
ED = 16
G = 16

NC = 2
NS = 16
NW = NC * NS
L = 16

N_PAD = 10240
DUMMY = 10008
ROWS_PER_TILE = N_PAD // NS

E_F = E + N
EPT = 10368
E_FP = NW * EPT
C = 32
NCHUNK = EPT // C

EA_PER_TILE = 10240
E_AP = NW * EA_PER_TILE
EA_CHUNK = 1024

_SC_PARAMS = dict(compiler_params=pltpu.CompilerParams(
    use_tc_tiling_on_sc=False, needs_layout_passes=False))


def _mesh():
    return plsc.VectorSubcoreMesh(
        core_axis_name="c", subcore_axis_name="s", num_cores=NC,
        num_subcores=NS)


def _edge_stats_body(ea_hbm, dst_hbm, out_hbm, acc, ea_buf, sbuf, dbuf,
                     idx_buf, sem):
    c = lax.axis_index("c")
    s = lax.axis_index("s")
    w = c * NS + s
    zero16 = jnp.zeros((L,), jnp.float32)
    one16 = jnp.ones((L,), jnp.float32)

    def _zero_row(i, _):
        ea_buf[i, pl.ds(0, L)] = zero16
        ea_buf[i, pl.ds(L, L)] = zero16
        return 0
    lax.fori_loop(0, EA_CHUNK, _zero_row, 0)

    row0 = pl.multiple_of(s * ROWS_PER_TILE, 8)
    pltpu.sync_copy(ea_buf.at[pl.ds(0, ROWS_PER_TILE)],
                    acc.at[pl.ds(row0, ROWS_PER_TILE)])

    def _mark_row(i, _):
        ea_buf[i, pl.ds(L, L)] = one16
        return 0
    lax.fori_loop(0, EA_CHUNK, _mark_row, 0)
    plsc.subcore_barrier()

    base_w = w * EA_PER_TILE

    def _chunk(b, _):
        base = pl.multiple_of(base_w + b * EA_CHUNK, 8)
        pltpu.sync_copy(ea_hbm.at[pl.ds(base * ED, EA_CHUNK * ED)], sbuf)
        pltpu.sync_copy(dst_hbm.at[pl.ds(base, EA_CHUNK)], dbuf)

        def _fill_row(r, _):
            ea_buf[r, pl.ds(0, ED)] = sbuf[pl.ds(r * ED, ED)]
            return 0
        lax.fori_loop(0, EA_CHUNK, _fill_row, 0)

        def _fill_idx(t, _):
            idx_buf[t >> 3, pl.ds((t & 7) * L, L)] = dbuf[pl.ds(t * L, L)]
            return 0
        lax.fori_loop(0, EA_CHUNK // L, _fill_idx, 0)

        for g in range(EA_CHUNK // 128):
            pltpu.sync_copy(
                ea_buf.at[pl.ds(g * 128, 128)],
                acc.at[idx_buf.at[g]],
                add=True)
        return 0

    lax.fori_loop(0, EA_PER_TILE // EA_CHUNK, _chunk, 0)
    plsc.subcore_barrier()
    pltpu.sync_copy(acc.at[pl.ds(row0, ROWS_PER_TILE)],
                    out_hbm.at[c, pl.ds(row0, ROWS_PER_TILE)])


def _edge_stats(ea_pad, dst_pad):
    k = pl.kernel(
        _edge_stats_body,
        out_type=jax.ShapeDtypeStruct((NC, N_PAD, 2 * ED), jnp.float32),
        mesh=_mesh(),
        scratch_types=[
            pltpu.VMEM_SHARED((N_PAD, 2 * ED), jnp.float32),
            pltpu.VMEM((EA_CHUNK, 2 * ED), jnp.float32),
            pltpu.VMEM((EA_CHUNK * ED,), jnp.float32),
            pltpu.VMEM((EA_CHUNK,), jnp.int32),
            pltpu.VMEM((EA_CHUNK // 128, 128), jnp.int32),
            pltpu.SemaphoreType.DMA,
        ],
        **_SC_PARAMS,
    )
    return k(ea_pad, dst_pad)


def _lane_bcast(v, j):
    idx = jnp.broadcast_to(j, (L,)).astype(jnp.int32)
    dn = lax.GatherDimensionNumbers(
        offset_dims=(), collapsed_slice_dims=(0,), start_index_map=(0,))
    return lax.gather(v, idx[:, None], dn, (1,),
                      mode=lax.GatherScatterMode.PROMISE_IN_BOUNDS)


def _make_layer_body(NH):
    def body(*refs):
        (srcf, dstf), p = refs[:2], 2
        a_h, p = refs[p:p + NH], p + NH
        b_h, p = refs[p:p + NH], p + NH
        e_h, p = refs[p:p + NH], p + NH
        (att_hbm, out_hbm, den_hbm, ex_hbm), p = refs[p:p + 4], p + 4
        (acc, ba0, bb0, be0, ba1, bb1, be1,
         exloc, denloc, attbuf, sidx0, sidx1, didx0, didx1, didx2,
         sem) = refs[p:]
        bufa = [ba0, ba1]
        bufb = [bb0, bb1]
        bufe = [be0, be1]
        sidx = [sidx0, sidx1]
        didx = [didx0, didx1]

        c = lax.axis_index("c")
        s = lax.axis_index("s")
        w = c * NS + s
        ebase = w * EPT
        zero16 = jnp.zeros((L,), jnp.float32)
        lane = lax.broadcasted_iota(jnp.int32, (L,), 0)

        pltpu.sync_copy(att_hbm, attbuf)
        attv = [[attbuf[pl.ds((h * 8 + k) * L, L)] for k in range(8)]
                for h in range(NH)]

        def _zero_buf(r, _):
            for c8 in range(8):
                ba0[r, pl.ds(c8 * L, L)] = zero16
            return 0

        row0 = pl.multiple_of(s * ROWS_PER_TILE, 8)

        def _zero_acc():
            lax.fori_loop(0, C, _zero_buf, 0)
            for z in range(ROWS_PER_TILE // C):
                pltpu.sync_copy(ba0, acc.at[pl.ds(row0 + z * C, C)])

        _zero_acc()

        def _zero_den(i, _):
            denloc[pl.ds(i * L, L)] = zero16
            return 0
        lax.fori_loop(0, N_PAD // L, _zero_den, 0)
        plsc.subcore_barrier()

        def _chunk_base(i):
            return pl.multiple_of(ebase + i * C, 8)

        def _pipelined(fire, drain, compute):
            fire(0, 0)
            fire(1, 1)

            def _pair(k, _):
                i0 = 2 * k
                drain(0)
                compute(i0, 0)
                fire(lax.rem(i0 + 2, NCHUNK), 0)
                drain(1)
                compute(i0 + 1, 1)
                fire(lax.rem(i0 + 3, NCHUNK), 1)
                return 0
            lax.fori_loop(0, NCHUNK // 2, _pair, 0)
            drain(0)
            drain(1)

        def _fire_half(h, with_b):
            def fire(i, st):
                base = _chunk_base(i)
                pltpu.sync_copy(srcf.at[pl.ds(base, C)], sidx[st])
                pltpu.async_copy(a_h[h].at[sidx[st]], bufa[st], sem)
                if with_b:
                    pltpu.sync_copy(dstf.at[pl.ds(base, C)], didx[st])
                    pltpu.async_copy(b_h[h].at[didx[st]], bufb[st], sem)
                    pltpu.async_copy(
                        e_h[h].at[pl.ds(base, C), :], bufe[st], sem)
                elif with_b is None:
                    pltpu.sync_copy(dstf.at[pl.ds(base, C)], didx[st])
            return fire

        def _drain_half(h, with_b):
            def drain(st):
                pltpu.make_async_copy(
                    a_h[h].at[sidx[st]], bufa[st], sem).wait()
                if with_b:
                    pltpu.make_async_copy(
                        b_h[h].at[didx[st]], bufb[st], sem).wait()
                    pltpu.make_async_copy(
                        e_h[h].at[pl.ds(0, C), :], bufe[st], sem).wait()
            return drain

        def _fill_didx2(st):
            def f(t, _):
                didx2[0, pl.ds(t * L, L)] = didx[st][pl.ds(t * L, L)]
                return 0
            lax.fori_loop(0, C // L, f, 0)

        def _partial_alpha(st, r, h):
            acc16 = zero16
            for c8 in range(8):
                sl = pl.ds(c8 * L, L)
                sv = bufa[st][r, sl] + bufb[st][r, sl] + bufe[st][r, sl]
                acc16 = acc16 + attv[h][c8] * jnp.maximum(sv, 0.2 * sv)
            return acc16

        if NH == 2:
            def _compute_a1(i, st):
                def _group(g, _):
                    def _e4(q, pa16):
                        for j4 in range(4):
                            j = q * 4 + j4
                            r = g * L + j
                            tot = jnp.sum(_partial_alpha(st, r, 1))
                            pa16 = jnp.where(
                                lane == j, jnp.broadcast_to(tot, (L,)), pa16)
                        return pa16
                    pa16 = lax.fori_loop(0, 4, _e4, zero16)
                    exloc[pl.ds(i * C + g * L, L)] = pa16
                    return 0
                lax.fori_loop(0, C // L, _group, 0)

            _pipelined(_fire_half(1, True), _drain_half(1, True),
                       _compute_a1)

        def _compute_a2(i, st):
            _fill_didx2(st)

            def _group(g, _):
                if NH == 2:
                    pa16 = exloc[pl.ds(i * C + g * L, L)]

                def _e4(q, ex16):
                    for j4 in range(4):
                        j = q * 4 + j4
                        r = g * L + j
                        tot = jnp.broadcast_to(
                            jnp.sum(_partial_alpha(st, r, 0)), (L,))
                        if NH == 2:
                            tot = tot + _lane_bcast(pa16, j)
                        exj = jnp.exp(tot)
                        for c8 in range(8):
                            sl = pl.ds(c8 * L, L)
                            bufa[st][r, sl] = bufa[st][r, sl] * exj
                        ex16 = jnp.where(lane == j, exj, ex16)
                    return ex16
                ex16 = lax.fori_loop(0, 4, _e4, zero16)

                dst16 = didx[st][pl.ds(g * L, L)]
                plsc.addupdate_scatter(denloc, [dst16], ex16)
                exloc[pl.ds(i * C + g * L, L)] = ex16
                return 0

            lax.fori_loop(0, C // L, _group, 0)
            pltpu.sync_copy(bufa[st], acc.at[didx2.at[0]], add=True)

        _pipelined(_fire_half(0, True), _drain_half(0, True), _compute_a2)

        pltpu.sync_copy(denloc, den_hbm.at[w])
        pltpu.sync_copy(exloc, ex_hbm.at[pl.ds(ebase, EPT)])
        plsc.subcore_barrier()
        pltpu.sync_copy(acc.at[pl.ds(row0, ROWS_PER_TILE)],
                        out_hbm.at[c, 0, pl.ds(row0, ROWS_PER_TILE)])

        if NH == 2:
            _zero_acc()
            plsc.subcore_barrier()

            def _compute_d(i, st):
                _fill_didx2(st)

                def _group(g, _):
                    exg = exloc[pl.ds(i * C + g * L, L)]

                    def _e4(q, _2):
                        for j4 in range(4):
                            j = q * 4 + j4
                            r = g * L + j
                            e_b = _lane_bcast(exg, j)
                            for c8 in range(8):
                                sl = pl.ds(c8 * L, L)
                                bufa[st][r, sl] = bufa[st][r, sl] * e_b
                        return 0
                    lax.fori_loop(0, 4, _e4, 0)
                    return 0

                lax.fori_loop(0, C // L, _group, 0)
                pltpu.sync_copy(bufa[st], acc.at[didx2.at[0]], add=True)

            _pipelined(_fire_half(1, None), _drain_half(1, False),
                       _compute_d)
            plsc.subcore_barrier()
            pltpu.sync_copy(acc.at[pl.ds(row0, ROWS_PER_TILE)],
                            out_hbm.at[c, 1, pl.ds(row0, ROWS_PER_TILE)])
    return body


def _gat_edge_phase(srcf, dstf, a_halves, b_halves, e_halves, att):
    NH = len(a_halves)
    k = pl.kernel(
        _make_layer_body(NH),
        out_type=[
            jax.ShapeDtypeStruct((NC, NH, N_PAD, 128), jnp.float32),
            jax.ShapeDtypeStruct((NW, N_PAD), jnp.float32),
            jax.ShapeDtypeStruct((E_FP,), jnp.float32),
        ],
        mesh=_mesh(),
        scratch_types=(
            [pltpu.VMEM_SHARED((N_PAD, 128), jnp.float32)]
            + [pltpu.VMEM((C, 128), jnp.float32)] * 6
            + [
                pltpu.VMEM((EPT,), jnp.float32),
                pltpu.VMEM((N_PAD,), jnp.float32),
                pltpu.VMEM((NH * 128,), jnp.float32),
                pltpu.VMEM((C,), jnp.int32),
                pltpu.VMEM((C,), jnp.int32),
                pltpu.VMEM((C,), jnp.int32),
                pltpu.VMEM((C,), jnp.int32),
                pltpu.VMEM((1, C), jnp.int32),
                pltpu.SemaphoreType.DMA,
            ]),
        **_SC_PARAMS,
    )
    return k(srcf, dstf, *a_halves, *b_halves, *e_halves, att)


C3 = 1296


def _attw_body(ex_hbm, dst_hbm, den_hbm, aw_hbm, dbuf, tmp, exbuf, dstbuf,
               awbuf, sem):
    c = lax.axis_index("c")
    s = lax.axis_index("s")
    w = c * NS + s
    ebase = w * EPT

    pltpu.sync_copy(den_hbm.at[0], dbuf)

    def _accum(t, _):
        pltpu.sync_copy(den_hbm.at[t], tmp)

        def _add(i, _2):
            sl = pl.ds(i * L, L)
            dbuf[sl] = dbuf[sl] + tmp[sl]
            return 0
        lax.fori_loop(0, N_PAD // L, _add, 0)
        return 0
    lax.fori_loop(1, NW, _accum, 0)

    def _chunk(i, _):
        base = pl.multiple_of(ebase + i * C3, 8)
        pltpu.sync_copy(ex_hbm.at[pl.ds(base, C3)], exbuf)
        pltpu.sync_copy(dst_hbm.at[pl.ds(base, C3)], dstbuf)

        def _group(g, _2):
            sl = pl.ds(g * L, L)
            d16 = plsc.load_gather(dbuf, [dstbuf[sl]])
            awbuf[sl] = exbuf[sl] / d16
            return 0
        lax.fori_loop(0, C3 // L, _group, 0)
        pltpu.sync_copy(awbuf, aw_hbm.at[pl.ds(base, C3)])
        return 0
    lax.fori_loop(0, EPT // C3, _chunk, 0)


def _attw(ex, dstf, den):
    k = pl.kernel(
        _attw_body,
        out_type=jax.ShapeDtypeStruct((E_FP,), jnp.float32),
        mesh=_mesh(),
        scratch_types=[
            pltpu.VMEM((N_PAD,), jnp.float32),
            pltpu.VMEM((N_PAD,), jnp.float32),
            pltpu.VMEM((C3,), jnp.float32),
            pltpu.VMEM((C3,), jnp.int32),
            pltpu.VMEM((C3,), jnp.float32),
            pltpu.SemaphoreType.DMA,
        ],
        **_SC_PARAMS,
    )
    return k(ex, dstf, den)


EBLK = 2048


def _e_proj(ea8, w8s):
    n_out = len(w8s)
    blk8 = EBLK // 8

    def body(x_ref, *refs):
        w_refs = refs[:n_out]
        o_refs = refs[n_out:]
        x = x_ref[...]
        for k in range(n_out):
            o_refs[k][...] = jnp.dot(
                x, w_refs[k][...], preferred_element_type=jnp.float32)

    grid = (E_FP // EBLK,)
    return pl.pallas_call(
        body,
        grid=grid,
        in_specs=[pl.BlockSpec((blk8, 128), lambda i: (i, 0))]
        + [pl.BlockSpec((128, 1024), lambda i: (0, 0))] * n_out,
        out_specs=[pl.BlockSpec((blk8, 1024), lambda i: (i, 0))] * n_out,
        out_shape=[jax.ShapeDtypeStruct((E_FP // 8, 1024), jnp.float32)] * n_out,
    )(ea8, *w8s)


def _x_proj(xp, ws):
    n_out = len(ws)
    K = xp.shape[1]

    def body(x_ref, *refs):
        w_refs = refs[:n_out]
        o_refs = refs[n_out:]
        x = x_ref[...]
        for k in range(n_out):
            o_refs[k][...] = jnp.dot(
                x, w_refs[k][...], preferred_element_type=jnp.float32)

    grid = (N_PAD // 256,)
    return pl.pallas_call(
        body,
        grid=grid,
        in_specs=[pl.BlockSpec((256, K), lambda i: (i, 0))]
        + [pl.BlockSpec((K, 128), lambda i: (0, 0))] * n_out,
        out_specs=[pl.BlockSpec((256, 128), lambda i: (i, 0))] * n_out,
        out_shape=[jax.ShapeDtypeStruct((N_PAD, 128), jnp.float32)] * n_out,
    )(xp, *ws)


def _finalize(out_p, den_p, bias_t, ws, relu):
    NH = out_p.shape[1]
    dout = NH * 128
    n_out = len(ws)

    def body(o_ref, d_ref, b_ref, *refs):
        w_refs = refs[:n_out]
        res = refs[n_out:]
        h_ref = res[0]
        o_res = res[1:]
        den = jnp.sum(d_ref[...], axis=0)
        halves = [o_ref[0, h] + o_ref[1, h] for h in range(NH)]
        hcat = jnp.concatenate(halves, axis=1) if NH > 1 else halves[0]
        fin = hcat / den[:, None] + b_ref[0:1, :]
        if relu:
            fin = jnp.maximum(fin, 0.0)
        row = (pl.program_id(0) * 256
               + lax.broadcasted_iota(jnp.int32, (256, 1), 0))
        fin = jnp.where(row < N, fin, 0.0)
        h_ref[...] = fin
        for k in range(n_out):
            o_res[k][...] = jnp.dot(
                fin, w_refs[k][...], preferred_element_type=jnp.float32)

    grid = (N_PAD // 256,)
    return pl.pallas_call(
        body,
        grid=grid,
        in_specs=[
            pl.BlockSpec((NC, NH, 256, 128), lambda i: (0, 0, i, 0)),
            pl.BlockSpec((NW, 256), lambda i: (0, i)),
            pl.BlockSpec((8, dout), lambda i: (0, 0)),
        ] + [pl.BlockSpec((dout, 128), lambda i: (0, 0))] * n_out,
        out_specs=[pl.BlockSpec((256, dout), lambda i: (i, 0))]
        + [pl.BlockSpec((256, 128), lambda i: (i, 0))] * n_out,
        out_shape=[jax.ShapeDtypeStruct((N_PAD, dout), jnp.float32)]
        + [jax.ShapeDtypeStruct((N_PAD, 128), jnp.float32)] * n_out,
    )(out_p, den_p, bias_t, *ws)


def _pool(h3, batchf):
    def body(h_ref, b_ref, emb_ref, gs_ref, gc_ref):
        i = pl.program_id(0)

        @pl.when(i == 0)
        def _init():
            gs_ref[...] = jnp.zeros_like(gs_ref)
            gc_ref[...] = jnp.zeros_like(gc_ref)

        b_ = b_ref[...]
        gid = lax.broadcasted_iota(jnp.int32, (256, G), 1)
        oh = (b_[:, None] == gid).astype(jnp.float32)
        gs_ref[...] += jnp.dot(oh.T, h_ref[...],
                               preferred_element_type=jnp.float32)
        gc_ref[...] += jnp.sum(oh, axis=0, keepdims=True)

        @pl.when(i == N_PAD // 256 - 1)
        def _fin():
            cnt = jnp.maximum(gc_ref[0, :], 1.0)
            emb_ref[...] = gs_ref[...] / cnt[:, None]

    grid = (N_PAD // 256,)
    return pl.pallas_call(
        body,
        grid=grid,
        in_specs=[
            pl.BlockSpec((256, 256), lambda i: (i, 0)),
            pl.BlockSpec((256,), lambda i: (i,)),
        ],
        out_specs=pl.BlockSpec((G, 256), lambda i: (0, 0)),
        out_shape=jax.ShapeDtypeStruct((G, 256), jnp.float32),
        scratch_shapes=[
            pltpu.VMEM((G, 256), jnp.float32),
            pltpu.VMEM((1, G), jnp.float32),
        ],
    )(h3, batchf)


def _kron8(w):
    return jnp.kron(jnp.eye(8, dtype=w.dtype), w)


def kernel(x, edge_index, edge_attr, batch, Wl1, Wr1, We1, att1, b1, Wl2, Wr2, We2, att2, b2, Wl3, Wr3, We3, att3, b3):
    src, dst = edge_index[0], edge_index[1]
    loop = jnp.arange(N, dtype=src.dtype)
    src_f = jnp.concatenate([src, loop])
    dst_f = jnp.concatenate([dst, loop])
    ei_ret = jnp.stack([src_f, dst_f])

    pad = E_AP - E
    ea_pad = jnp.concatenate(
        [edge_attr, jnp.zeros((pad, ED), jnp.float32)], axis=0).reshape(-1)
    dst_pad = jnp.concatenate(
        [dst.astype(jnp.int32), jnp.full((pad,), DUMMY, jnp.int32)])
    stats = _edge_stats(ea_pad, dst_pad)
    stats = stats[0] + stats[1]
    sums = stats[:N, :ED]
    cnt = stats[:N, ED]
    loop_attr = sums / jnp.clip(cnt, 1.0)[:, None]

    epad = E_FP - E_F
    srcf_p = jnp.concatenate(
        [src_f.astype(jnp.int32), jnp.zeros((epad,), jnp.int32)])
    dstf_p = jnp.concatenate(
        [dst_f.astype(jnp.int32), jnp.full((epad,), DUMMY, jnp.int32)])
    eaf_p = jnp.concatenate(
        [edge_attr, loop_attr, jnp.zeros((epad, ED), jnp.float32)], axis=0)
    ea8 = eaf_p.reshape(E_FP // 8, 8 * ED)

    w8s = [_kron8(We1), _kron8(We2[:, :128]), _kron8(We2[:, 128:]),
           _kron8(We3[:, :128]), _kron8(We3[:, 128:])]
    e1, e2L, e2H, e3L, e3H = [
        o.reshape(E_FP, 128) for o in _e_proj(ea8, w8s)]

    xp = jnp.concatenate(
        [x, jnp.zeros((N_PAD - N, x.shape[1]), jnp.float32)], axis=0)
    a1, bm1 = _x_proj(xp, [Wl1, Wr1])

    out1, den1, _ = _gat_edge_phase(srcf_p, dstf_p, [a1], [bm1], [e1], att1)
    b1t = jnp.tile(b1[None, :], (8, 1))
    _, a2L, a2H, b2L, b2H = _finalize(
        out1, den1, b1t, [Wl2[:, :128], Wl2[:, 128:],
                          Wr2[:, :128], Wr2[:, 128:]], relu=True)

    out2, den2, _ = _gat_edge_phase(
        srcf_p, dstf_p, [a2L, a2H], [b2L, b2H], [e2L, e2H], att2)
    b2t = jnp.tile(b2[None, :], (8, 1))
    _, a3L, a3H, b3L, b3H = _finalize(
        out2, den2, b2t, [Wl3[:, :128], Wl3[:, 128:],
                          Wr3[:, :128], Wr3[:, 128:]], relu=True)

    out3, den3, ex3 = _gat_edge_phase(
        srcf_p, dstf_p, [a3L, a3H], [b3L, b3H], [e3L, e3H], att3)
    b3t = jnp.tile(b3[None, :], (8, 1))
    (h3,) = _finalize(out3, den3, b3t, [], relu=False)

    batchf = jnp.concatenate(
        [batch.astype(jnp.int32), jnp.full((N_PAD - N,), G, jnp.int32)])
    emb = _pool(h3, batchf)

    aw = _attw(ex3, dstf_p, den3)
    att_w = aw[:E_F]

    return emb, ei_ret, att_w


# speedup vs baseline: 1.0001x; 1.0001x over previous
import jax
import jax.numpy as jnp
from jax import lax
from jax.experimental import pallas as pl
from jax.experimental.pallas import tpu as pltpu
import jax.experimental.pallas.tpu_sc as plsc

N = 10000
E = 320000
ED = 16
G = 16

NC = 2
NS = 16
NW = NC * NS
L = 16

N_PAD = 10240
DUMMY = 10008
ROWS_PER_TILE = N_PAD // NS

E_F = E + N
EPT = 10368
E_FP = NW * EPT
C = 32
NCHUNK = EPT // C

EA_PER_TILE = 10240
E_AP = NW * EA_PER_TILE
EA_CHUNK = 1024

_SC_PARAMS = dict(compiler_params=pltpu.CompilerParams(
    use_tc_tiling_on_sc=False, needs_layout_passes=False))


def _mesh():
    return plsc.VectorSubcoreMesh(
        core_axis_name="c", subcore_axis_name="s", num_cores=NC,
        num_subcores=NS)


def _edge_stats_body(ea_hbm, dst_hbm, out_hbm, acc, ea_buf, sbuf, dbuf,
                     idx_buf, sem):
    c = lax.axis_index("c")
    s = lax.axis_index("s")
    w = c * NS + s
    zero16 = jnp.zeros((L,), jnp.float32)
    one16 = jnp.ones((L,), jnp.float32)

    def _zero_row(i, _):
        ea_buf[i, pl.ds(0, L)] = zero16
        ea_buf[i, pl.ds(L, L)] = zero16
        return 0
    lax.fori_loop(0, EA_CHUNK, _zero_row, 0)

    row0 = pl.multiple_of(s * ROWS_PER_TILE, 8)
    pltpu.sync_copy(ea_buf.at[pl.ds(0, ROWS_PER_TILE)],
                    acc.at[pl.ds(row0, ROWS_PER_TILE)])

    def _mark_row(i, _):
        ea_buf[i, pl.ds(L, L)] = one16
        return 0
    lax.fori_loop(0, EA_CHUNK, _mark_row, 0)
    plsc.subcore_barrier()

    base_w = w * EA_PER_TILE

    def _chunk(b, _):
        base = pl.multiple_of(base_w + b * EA_CHUNK, 8)
        pltpu.sync_copy(ea_hbm.at[pl.ds(base * ED, EA_CHUNK * ED)], sbuf)
        pltpu.sync_copy(dst_hbm.at[pl.ds(base, EA_CHUNK)], dbuf)

        def _fill_row(r, _):
            ea_buf[r, pl.ds(0, ED)] = sbuf[pl.ds(r * ED, ED)]
            return 0
        lax.fori_loop(0, EA_CHUNK, _fill_row, 0)

        def _fill_idx(t, _):
            idx_buf[t >> 3, pl.ds((t & 7) * L, L)] = dbuf[pl.ds(t * L, L)]
            return 0
        lax.fori_loop(0, EA_CHUNK // L, _fill_idx, 0)

        for g in range(EA_CHUNK // 128):
            pltpu.sync_copy(
                ea_buf.at[pl.ds(g * 128, 128)],
                acc.at[idx_buf.at[g]],
                add=True)
        return 0

    lax.fori_loop(0, EA_PER_TILE // EA_CHUNK, _chunk, 0)
    plsc.subcore_barrier()
    pltpu.sync_copy(acc.at[pl.ds(row0, ROWS_PER_TILE)],
                    out_hbm.at[c, pl.ds(row0, ROWS_PER_TILE)])


def _edge_stats(ea_pad, dst_pad):
    k = pl.kernel(
        _edge_stats_body,
        out_type=jax.ShapeDtypeStruct((NC, N_PAD, 2 * ED), jnp.float32),
        mesh=_mesh(),
        scratch_types=[
            pltpu.VMEM_SHARED((N_PAD, 2 * ED), jnp.float32),
            pltpu.VMEM((EA_CHUNK, 2 * ED), jnp.float32),
            pltpu.VMEM((EA_CHUNK * ED,), jnp.float32),
            pltpu.VMEM((EA_CHUNK,), jnp.int32),
            pltpu.VMEM((EA_CHUNK // 128, 128), jnp.int32),
            pltpu.SemaphoreType.DMA,
        ],
        **_SC_PARAMS,
    )
    return k(ea_pad, dst_pad)


def _lane_bcast(v, j):
    idx = jnp.broadcast_to(j, (L,)).astype(jnp.int32)
    dn = lax.GatherDimensionNumbers(
        offset_dims=(), collapsed_slice_dims=(0,), start_index_map=(0,))
    return lax.gather(v, idx[:, None], dn, (1,),
                      mode=lax.GatherScatterMode.PROMISE_IN_BOUNDS)


def _make_layer_body(NH):
    def body(*refs):
        (srcf, dstf), p = refs[:2], 2
        a_h, p = refs[p:p + NH], p + NH
        b_h, p = refs[p:p + NH], p + NH
        e_h, p = refs[p:p + NH], p + NH
        (att_hbm, out_hbm, den_hbm, ex_hbm), p = refs[p:p + 4], p + 4
        (acc, ba0, bb0, be0, ba1, bb1, be1,
         exloc, denloc, attbuf, sidx0, sidx1, didx0, didx1, didx2,
         sem) = refs[p:]
        bufa = [ba0, ba1]
        bufb = [bb0, bb1]
        bufe = [be0, be1]
        sidx = [sidx0, sidx1]
        didx = [didx0, didx1]

        c = lax.axis_index("c")
        s = lax.axis_index("s")
        w = c * NS + s
        ebase = w * EPT
        zero16 = jnp.zeros((L,), jnp.float32)
        lane = lax.broadcasted_iota(jnp.int32, (L,), 0)

        pltpu.sync_copy(att_hbm, attbuf)
        attv = [[attbuf[pl.ds((h * 8 + k) * L, L)] for k in range(8)]
                for h in range(NH)]

        def _zero_buf(r, _):
            for c8 in range(8):
                ba0[r, pl.ds(c8 * L, L)] = zero16
            return 0

        row0 = pl.multiple_of(s * ROWS_PER_TILE, 8)

        def _zero_acc():
            lax.fori_loop(0, C, _zero_buf, 0)
            for z in range(ROWS_PER_TILE // C):
                pltpu.sync_copy(ba0, acc.at[pl.ds(row0 + z * C, C)])

        _zero_acc()

        def _zero_den(i, _):
            denloc[pl.ds(i * L, L)] = zero16
            return 0
        lax.fori_loop(0, N_PAD // L, _zero_den, 0)
        plsc.subcore_barrier()

        def _chunk_base(i):
            return pl.multiple_of(ebase + i * C, 8)

        def _pipelined(fire, drain, compute):
            fire(0, 0)
            fire(1, 1)

            def _pair(k, _):
                i0 = 2 * k
                drain(0)
                compute(i0, 0)
                fire(lax.rem(i0 + 2, NCHUNK), 0)
                drain(1)
                compute(i0 + 1, 1)
                fire(lax.rem(i0 + 3, NCHUNK), 1)
                return 0
            lax.fori_loop(0, NCHUNK // 2, _pair, 0)
            drain(0)
            drain(1)

        def _fire_half(h, with_b):
            def fire(i, st):
                base = _chunk_base(i)
                pltpu.sync_copy(srcf.at[pl.ds(base, C)], sidx[st])
                pltpu.async_copy(a_h[h].at[sidx[st]], bufa[st], sem)
                if with_b:
                    pltpu.sync_copy(dstf.at[pl.ds(base, C)], didx[st])
                    pltpu.async_copy(b_h[h].at[didx[st]], bufb[st], sem)
                    pltpu.async_copy(
                        e_h[h].at[pl.ds(base, C), :], bufe[st], sem)
                elif with_b is None:
                    pltpu.sync_copy(dstf.at[pl.ds(base, C)], didx[st])
            return fire

        def _drain_half(h, with_b):
            def drain(st):
                pltpu.make_async_copy(
                    a_h[h].at[sidx[st]], bufa[st], sem).wait()
                if with_b:
                    pltpu.make_async_copy(
                        b_h[h].at[didx[st]], bufb[st], sem).wait()
                    pltpu.make_async_copy(
                        e_h[h].at[pl.ds(0, C), :], bufe[st], sem).wait()
            return drain

        def _fill_didx2(st):
            def f(t, _):
                didx2[0, pl.ds(t * L, L)] = didx[st][pl.ds(t * L, L)]
                return 0
            lax.fori_loop(0, C // L, f, 0)

        def _partial_alpha(st, r, h):
            acc16 = zero16
            for c8 in range(8):
                sl = pl.ds(c8 * L, L)
                sv = bufa[st][r, sl] + bufb[st][r, sl] + bufe[st][r, sl]
                acc16 = acc16 + attv[h][c8] * jnp.maximum(sv, 0.2 * sv)
            return acc16

        if NH == 2:
            def _compute_a1(i, st):
                def _group(g, _):
                    def _e4(q, pa16):
                        for j4 in range(4):
                            j = q * 4 + j4
                            r = g * L + j
                            tot = jnp.sum(_partial_alpha(st, r, 1))
                            pa16 = jnp.where(
                                lane == j, jnp.broadcast_to(tot, (L,)), pa16)
                        return pa16
                    pa16 = lax.fori_loop(0, 4, _e4, zero16)
                    exloc[pl.ds(i * C + g * L, L)] = pa16
                    return 0
                lax.fori_loop(0, C // L, _group, 0)

            _pipelined(_fire_half(1, True), _drain_half(1, True),
                       _compute_a1)

        def _compute_a2(i, st):
            _fill_didx2(st)

            def _group(g, _):
                if NH == 2:
                    pa16 = exloc[pl.ds(i * C + g * L, L)]

                def _e4(q, ex16):
                    for j4 in range(4):
                        j = q * 4 + j4
                        r = g * L + j
                        avals = []
                        acc16 = zero16
                        for c8 in range(8):
                            sl = pl.ds(c8 * L, L)
                            av = bufa[st][r, sl]
                            avals.append(av)
                            sv = av + bufb[st][r, sl] + bufe[st][r, sl]
                            acc16 = acc16 + attv[0][c8] * jnp.maximum(
                                sv, 0.2 * sv)
                        tot = jnp.broadcast_to(jnp.sum(acc16), (L,))
                        if NH == 2:
                            tot = tot + _lane_bcast(pa16, j)
                        exj = jnp.exp(tot)
                        for c8 in range(8):
                            sl = pl.ds(c8 * L, L)
                            bufa[st][r, sl] = avals[c8] * exj
                        ex16 = jnp.where(lane == j, exj, ex16)
                    return ex16
                ex16 = lax.fori_loop(0, 4, _e4, zero16)

                dst16 = didx[st][pl.ds(g * L, L)]
                plsc.addupdate_scatter(denloc, [dst16], ex16)
                exloc[pl.ds(i * C + g * L, L)] = ex16
                return 0

            lax.fori_loop(0, C // L, _group, 0)
            pltpu.sync_copy(bufa[st], acc.at[didx2.at[0]], add=True)

        _pipelined(_fire_half(0, True), _drain_half(0, True), _compute_a2)

        pltpu.sync_copy(denloc, den_hbm.at[w])
        pltpu.sync_copy(exloc, ex_hbm.at[pl.ds(ebase, EPT)])
        plsc.subcore_barrier()
        pltpu.sync_copy(acc.at[pl.ds(row0, ROWS_PER_TILE)],
                        out_hbm.at[c, 0, pl.ds(row0, ROWS_PER_TILE)])

        if NH == 2:
            _zero_acc()
            plsc.subcore_barrier()

            def _compute_d(i, st):
                _fill_didx2(st)

                def _group(g, _):
                    exg = exloc[pl.ds(i * C + g * L, L)]

                    def _e4(q, _2):
                        for j4 in range(4):
                            j = q * 4 + j4
                            r = g * L + j
                            e_b = _lane_bcast(exg, j)
                            for c8 in range(8):
                                sl = pl.ds(c8 * L, L)
                                bufa[st][r, sl] = bufa[st][r, sl] * e_b
                        return 0
                    lax.fori_loop(0, 4, _e4, 0)
                    return 0

                lax.fori_loop(0, C // L, _group, 0)
                pltpu.sync_copy(bufa[st], acc.at[didx2.at[0]], add=True)

            _pipelined(_fire_half(1, None), _drain_half(1, False),
                       _compute_d)
            plsc.subcore_barrier()
            pltpu.sync_copy(acc.at[pl.ds(row0, ROWS_PER_TILE)],
                            out_hbm.at[c, 1, pl.ds(row0, ROWS_PER_TILE)])
    return body


def _gat_edge_phase(srcf, dstf, a_halves, b_halves, e_halves, att):
    NH = len(a_halves)
    k = pl.kernel(
        _make_layer_body(NH),
        out_type=[
            jax.ShapeDtypeStruct((NC, NH, N_PAD, 128), jnp.float32),
            jax.ShapeDtypeStruct((NW, N_PAD), jnp.float32),
            jax.ShapeDtypeStruct((E_FP,), jnp.float32),
        ],
        mesh=_mesh(),
        scratch_types=(
            [pltpu.VMEM_SHARED((N_PAD, 128), jnp.float32)]
            + [pltpu.VMEM((C, 128), jnp.float32)] * 6
            + [
                pltpu.VMEM((EPT,), jnp.float32),
                pltpu.VMEM((N_PAD,), jnp.float32),
                pltpu.VMEM((NH * 128,), jnp.float32),
                pltpu.VMEM((C,), jnp.int32),
                pltpu.VMEM((C,), jnp.int32),
                pltpu.VMEM((C,), jnp.int32),
                pltpu.VMEM((C,), jnp.int32),
                pltpu.VMEM((1, C), jnp.int32),
                pltpu.SemaphoreType.DMA,
            ]),
        **_SC_PARAMS,
    )
    return k(srcf, dstf, *a_halves, *b_halves, *e_halves, att)


C3 = 1296


def _attw_body(ex_hbm, dst_hbm, den_hbm, aw_hbm, dbuf, tmp, exbuf, dstbuf,
               awbuf, sem):
    c = lax.axis_index("c")
    s = lax.axis_index("s")
    w = c * NS + s
    ebase = w * EPT

    pltpu.sync_copy(den_hbm.at[0], dbuf)

    def _accum(t, _):
        pltpu.sync_copy(den_hbm.at[t], tmp)

        def _add(i, _2):
            sl = pl.ds(i * L, L)
            dbuf[sl] = dbuf[sl] + tmp[sl]
            return 0
        lax.fori_loop(0, N_PAD // L, _add, 0)
        return 0
    lax.fori_loop(1, NW, _accum, 0)

    def _chunk(i, _):
        base = pl.multiple_of(ebase + i * C3, 8)
        pltpu.sync_copy(ex_hbm.at[pl.ds(base, C3)], exbuf)
        pltpu.sync_copy(dst_hbm.at[pl.ds(base, C3)], dstbuf)

        def _group(g, _2):
            sl = pl.ds(g * L, L)
            d16 = plsc.load_gather(dbuf, [dstbuf[sl]])
            awbuf[sl] = exbuf[sl] / d16
            return 0
        lax.fori_loop(0, C3 // L, _group, 0)
        pltpu.sync_copy(awbuf, aw_hbm.at[pl.ds(base, C3)])
        return 0
    lax.fori_loop(0, EPT // C3, _chunk, 0)


def _attw(ex, dstf, den):
    k = pl.kernel(
        _attw_body,
        out_type=jax.ShapeDtypeStruct((E_FP,), jnp.float32),
        mesh=_mesh(),
        scratch_types=[
            pltpu.VMEM((N_PAD,), jnp.float32),
            pltpu.VMEM((N_PAD,), jnp.float32),
            pltpu.VMEM((C3,), jnp.float32),
            pltpu.VMEM((C3,), jnp.int32),
            pltpu.VMEM((C3,), jnp.float32),
            pltpu.SemaphoreType.DMA,
        ],
        **_SC_PARAMS,
    )
    return k(ex, dstf, den)


EBLK = 2048


def _e_proj(ea8, w8s):
    n_out = len(w8s)
    blk8 = EBLK // 8

    def body(x_ref, *refs):
        w_refs = refs[:n_out]
        o_refs = refs[n_out:]
        x = x_ref[...]
        for k in range(n_out):
            o_refs[k][...] = jnp.dot(
                x, w_refs[k][...], preferred_element_type=jnp.float32)

    grid = (E_FP // EBLK,)
    return pl.pallas_call(
        body,
        grid=grid,
        in_specs=[pl.BlockSpec((blk8, 128), lambda i: (i, 0))]
        + [pl.BlockSpec((128, 1024), lambda i: (0, 0))] * n_out,
        out_specs=[pl.BlockSpec((blk8, 1024), lambda i: (i, 0))] * n_out,
        out_shape=[jax.ShapeDtypeStruct((E_FP // 8, 1024), jnp.float32)] * n_out,
    )(ea8, *w8s)


def _x_proj(xp, ws):
    n_out = len(ws)
    K = xp.shape[1]

    def body(x_ref, *refs):
        w_refs = refs[:n_out]
        o_refs = refs[n_out:]
        x = x_ref[...]
        for k in range(n_out):
            o_refs[k][...] = jnp.dot(
                x, w_refs[k][...], preferred_element_type=jnp.float32)

    grid = (N_PAD // 256,)
    return pl.pallas_call(
        body,
        grid=grid,
        in_specs=[pl.BlockSpec((256, K), lambda i: (i, 0))]
        + [pl.BlockSpec((K, 128), lambda i: (0, 0))] * n_out,
        out_specs=[pl.BlockSpec((256, 128), lambda i: (i, 0))] * n_out,
        out_shape=[jax.ShapeDtypeStruct((N_PAD, 128), jnp.float32)] * n_out,
    )(xp, *ws)


def _finalize(out_p, den_p, bias_t, ws, relu):
    NH = out_p.shape[1]
    dout = NH * 128
    n_out = len(ws)

    def body(o_ref, d_ref, b_ref, *refs):
        w_refs = refs[:n_out]
        res = refs[n_out:]
        h_ref = res[0]
        o_res = res[1:]
        den = jnp.sum(d_ref[...], axis=0)
        halves = [o_ref[0, h] + o_ref[1, h] for h in range(NH)]
        hcat = jnp.concatenate(halves, axis=1) if NH > 1 else halves[0]
        fin = hcat / den[:, None] + b_ref[0:1, :]
        if relu:
            fin = jnp.maximum(fin, 0.0)
        row = (pl.program_id(0) * 256
               + lax.broadcasted_iota(jnp.int32, (256, 1), 0))
        fin = jnp.where(row < N, fin, 0.0)
        h_ref[...] = fin
        for k in range(n_out):
            o_res[k][...] = jnp.dot(
                fin, w_refs[k][...], preferred_element_type=jnp.float32)

    grid = (N_PAD // 256,)
    return pl.pallas_call(
        body,
        grid=grid,
        in_specs=[
            pl.BlockSpec((NC, NH, 256, 128), lambda i: (0, 0, i, 0)),
            pl.BlockSpec((NW, 256), lambda i: (0, i)),
            pl.BlockSpec((8, dout), lambda i: (0, 0)),
        ] + [pl.BlockSpec((dout, 128), lambda i: (0, 0))] * n_out,
        out_specs=[pl.BlockSpec((256, dout), lambda i: (i, 0))]
        + [pl.BlockSpec((256, 128), lambda i: (i, 0))] * n_out,
        out_shape=[jax.ShapeDtypeStruct((N_PAD, dout), jnp.float32)]
        + [jax.ShapeDtypeStruct((N_PAD, 128), jnp.float32)] * n_out,
    )(out_p, den_p, bias_t, *ws)


def _pool(h3, batchf):
    def body(h_ref, b_ref, emb_ref, gs_ref, gc_ref):
        i = pl.program_id(0)

        @pl.when(i == 0)
        def _init():
            gs_ref[...] = jnp.zeros_like(gs_ref)
            gc_ref[...] = jnp.zeros_like(gc_ref)

        b_ = b_ref[...]
        gid = lax.broadcasted_iota(jnp.int32, (256, G), 1)
        oh = (b_[:, None] == gid).astype(jnp.float32)
        gs_ref[...] += jnp.dot(oh.T, h_ref[...],
                               preferred_element_type=jnp.float32)
        gc_ref[...] += jnp.sum(oh, axis=0, keepdims=True)

        @pl.when(i == N_PAD // 256 - 1)
        def _fin():
            cnt = jnp.maximum(gc_ref[0, :], 1.0)
            emb_ref[...] = gs_ref[...] / cnt[:, None]

    grid = (N_PAD // 256,)
    return pl.pallas_call(
        body,
        grid=grid,
        in_specs=[
            pl.BlockSpec((256, 256), lambda i: (i, 0)),
            pl.BlockSpec((256,), lambda i: (i,)),
        ],
        out_specs=pl.BlockSpec((G, 256), lambda i: (0, 0)),
        out_shape=jax.ShapeDtypeStruct((G, 256), jnp.float32),
        scratch_shapes=[
            pltpu.VMEM((G, 256), jnp.float32),
            pltpu.VMEM((1, G), jnp.float32),
        ],
    )(h3, batchf)


def _kron8(w):
    return jnp.kron(jnp.eye(8, dtype=w.dtype), w)


def kernel(x, edge_index, edge_attr, batch, Wl1, Wr1, We1, att1, b1, Wl2, Wr2, We2, att2, b2, Wl3, Wr3, We3, att3, b3):
    src, dst = edge_index[0], edge_index[1]
    loop = jnp.arange(N, dtype=src.dtype)
    src_f = jnp.concatenate([src, loop])
    dst_f = jnp.concatenate([dst, loop])
    ei_ret = jnp.stack([src_f, dst_f])

    pad = E_AP - E
    ea_pad = jnp.concatenate(
        [edge_attr, jnp.zeros((pad, ED), jnp.float32)], axis=0).reshape(-1)
    dst_pad = jnp.concatenate(
        [dst.astype(jnp.int32), jnp.full((pad,), DUMMY, jnp.int32)])
    stats = _edge_stats(ea_pad, dst_pad)
    stats = stats[0] + stats[1]
    sums = stats[:N, :ED]
    cnt = stats[:N, ED]
    loop_attr = sums / jnp.clip(cnt, 1.0)[:, None]

    epad = E_FP - E_F
    srcf_p = jnp.concatenate(
        [src_f.astype(jnp.int32), jnp.zeros((epad,), jnp.int32)])
    dstf_p = jnp.concatenate(
        [dst_f.astype(jnp.int32), jnp.full((epad,), DUMMY, jnp.int32)])
    eaf_p = jnp.concatenate(
        [edge_attr, loop_attr, jnp.zeros((epad, ED), jnp.float32)], axis=0)
    ea8 = eaf_p.reshape(E_FP // 8, 8 * ED)

    w8s = [_kron8(We1), _kron8(We2[:, :128]), _kron8(We2[:, 128:]),
           _kron8(We3[:, :128]), _kron8(We3[:, 128:])]
    e1, e2L, e2H, e3L, e3H = [
        o.reshape(E_FP, 128) for o in _e_proj(ea8, w8s)]

    xp = jnp.concatenate(
        [x, jnp.zeros((N_PAD - N, x.shape[1]), jnp.float32)], axis=0)
    a1, bm1 = _x_proj(xp, [Wl1, Wr1])

    out1, den1, _ = _gat_edge_phase(srcf_p, dstf_p, [a1], [bm1], [e1], att1)
    b1t = jnp.tile(b1[None, :], (8, 1))
    _, a2L, a2H, b2L, b2H = _finalize(
        out1, den1, b1t, [Wl2[:, :128], Wl2[:, 128:],
                          Wr2[:, :128], Wr2[:, 128:]], relu=True)

    out2, den2, _ = _gat_edge_phase(
        srcf_p, dstf_p, [a2L, a2H], [b2L, b2H], [e2L, e2H], att2)
    b2t = jnp.tile(b2[None, :], (8, 1))
    _, a3L, a3H, b3L, b3H = _finalize(
        out2, den2, b2t, [Wl3[:, :128], Wl3[:, 128:],
                          Wr3[:, :128], Wr3[:, 128:]], relu=True)

    out3, den3, ex3 = _gat_edge_phase(
        srcf_p, dstf_p, [a3L, a3H], [b3L, b3H], [e3L, e3H], att3)
    b3t = jnp.tile(b3[None, :], (8, 1))
    (h3,) = _finalize(out3, den3, b3t, [], relu=False)

    batchf = jnp.concatenate(
        [batch.astype(jnp.int32), jnp.full((N_PAD - N,), G, jnp.int32)])
    emb = _pool(h3, batchf)

    aw = _attw(ex3, dstf_p, den3)
    att_w = aw[:E_F]

    return emb, ei_ret, att_w


# speedup vs baseline: 1.0221x; 1.0219x over previous
import jax
import jax.numpy as jnp
from jax import lax
from jax.experimental import pallas as pl
from jax.experimental.pallas import tpu as pltpu
import jax.experimental.pallas.tpu_sc as plsc

N = 10000
E = 320000
ED = 16
G = 16

NC = 2
NS = 16
NW = NC * NS
L = 16

N_PAD = 10240
DUMMY = 10008
ROWS_PER_TILE = N_PAD // NS

E_F = E + N
EPT = 10368
E_FP = NW * EPT
C = 32
NCHUNK = EPT // C

EA_PER_TILE = 10240
E_AP = NW * EA_PER_TILE
EA_CHUNK = 1024

_SC_PARAMS = dict(compiler_params=pltpu.CompilerParams(
    use_tc_tiling_on_sc=False, needs_layout_passes=False))


def _mesh():
    return plsc.VectorSubcoreMesh(
        core_axis_name="c", subcore_axis_name="s", num_cores=NC,
        num_subcores=NS)


def _edge_stats_body(ea_hbm, dst_hbm, out_hbm, acc, ea_buf, sbuf, dbuf,
                     idx_buf, sem):
    c = lax.axis_index("c")
    s = lax.axis_index("s")
    w = c * NS + s
    zero16 = jnp.zeros((L,), jnp.float32)
    one16 = jnp.ones((L,), jnp.float32)

    def _zero_row(i, _):
        ea_buf[i, pl.ds(0, L)] = zero16
        ea_buf[i, pl.ds(L, L)] = zero16
        return 0
    lax.fori_loop(0, EA_CHUNK, _zero_row, 0)

    row0 = pl.multiple_of(s * ROWS_PER_TILE, 8)
    pltpu.sync_copy(ea_buf.at[pl.ds(0, ROWS_PER_TILE)],
                    acc.at[pl.ds(row0, ROWS_PER_TILE)])

    def _mark_row(i, _):
        ea_buf[i, pl.ds(L, L)] = one16
        return 0
    lax.fori_loop(0, EA_CHUNK, _mark_row, 0)
    plsc.subcore_barrier()

    base_w = w * EA_PER_TILE

    def _chunk(b, _):
        base = pl.multiple_of(base_w + b * EA_CHUNK, 8)
        pltpu.sync_copy(ea_hbm.at[pl.ds(base * ED, EA_CHUNK * ED)], sbuf)
        pltpu.sync_copy(dst_hbm.at[pl.ds(base, EA_CHUNK)], dbuf)

        def _fill_row(r, _):
            ea_buf[r, pl.ds(0, ED)] = sbuf[pl.ds(r * ED, ED)]
            return 0
        lax.fori_loop(0, EA_CHUNK, _fill_row, 0)

        def _fill_idx(t, _):
            idx_buf[t >> 3, pl.ds((t & 7) * L, L)] = dbuf[pl.ds(t * L, L)]
            return 0
        lax.fori_loop(0, EA_CHUNK // L, _fill_idx, 0)

        for g in range(EA_CHUNK // 128):
            pltpu.sync_copy(
                ea_buf.at[pl.ds(g * 128, 128)],
                acc.at[idx_buf.at[g]],
                add=True)
        return 0

    lax.fori_loop(0, EA_PER_TILE // EA_CHUNK, _chunk, 0)
    plsc.subcore_barrier()
    pltpu.sync_copy(acc.at[pl.ds(row0, ROWS_PER_TILE)],
                    out_hbm.at[c, pl.ds(row0, ROWS_PER_TILE)])


def _edge_stats(ea_pad, dst_pad):
    k = pl.kernel(
        _edge_stats_body,
        out_type=jax.ShapeDtypeStruct((NC, N_PAD, 2 * ED), jnp.float32),
        mesh=_mesh(),
        scratch_types=[
            pltpu.VMEM_SHARED((N_PAD, 2 * ED), jnp.float32),
            pltpu.VMEM((EA_CHUNK, 2 * ED), jnp.float32),
            pltpu.VMEM((EA_CHUNK * ED,), jnp.float32),
            pltpu.VMEM((EA_CHUNK,), jnp.int32),
            pltpu.VMEM((EA_CHUNK // 128, 128), jnp.int32),
            pltpu.SemaphoreType.DMA,
        ],
        **_SC_PARAMS,
    )
    return k(ea_pad, dst_pad)


def _lane_bcast(v, j):
    idx = jnp.broadcast_to(j, (L,)).astype(jnp.int32)
    dn = lax.GatherDimensionNumbers(
        offset_dims=(), collapsed_slice_dims=(0,), start_index_map=(0,))
    return lax.gather(v, idx[:, None], dn, (1,),
                      mode=lax.GatherScatterMode.PROMISE_IN_BOUNDS)


def _make_layer_body(NH):
    def body(*refs):
        (srcf, dstf), p = refs[:2], 2
        a_h, p = refs[p:p + NH], p + NH
        b_h, p = refs[p:p + NH], p + NH
        e_h, p = refs[p:p + NH], p + NH
        (att_hbm, out_hbm, den_hbm, ex_hbm), p = refs[p:p + 4], p + 4
        (acc, ba0, bb0, be0, ba1, bb1, be1,
         exloc, denloc, attbuf, tbuf, sidx0, sidx1, didx0, didx1, didx2,
         sem) = refs[p:]
        bufa = [ba0, ba1]
        bufb = [bb0, bb1]
        bufe = [be0, be1]
        sidx = [sidx0, sidx1]
        didx = [didx0, didx1]

        c = lax.axis_index("c")
        s = lax.axis_index("s")
        w = c * NS + s
        ebase = w * EPT
        zero16 = jnp.zeros((L,), jnp.float32)
        lane = lax.broadcasted_iota(jnp.int32, (L,), 0)

        pltpu.sync_copy(att_hbm, attbuf)
        attv = [[attbuf[pl.ds((h * 8 + k) * L, L)] for k in range(8)]
                for h in range(NH)]

        def _zero_buf(r, _):
            for c8 in range(8):
                ba0[r, pl.ds(c8 * L, L)] = zero16
            return 0

        row0 = pl.multiple_of(s * ROWS_PER_TILE, 8)

        def _zero_acc():
            lax.fori_loop(0, C, _zero_buf, 0)
            for z in range(ROWS_PER_TILE // C):
                pltpu.sync_copy(ba0, acc.at[pl.ds(row0 + z * C, C)])

        _zero_acc()

        def _zero_den(i, _):
            denloc[pl.ds(i * L, L)] = zero16
            return 0
        lax.fori_loop(0, N_PAD // L, _zero_den, 0)
        plsc.subcore_barrier()

        def _chunk_base(i):
            return pl.multiple_of(ebase + i * C, 8)

        def _pipelined(fire, drain, compute):
            fire(0, 0)
            fire(1, 1)

            def _pair(k, _):
                i0 = 2 * k
                drain(0)
                compute(i0, 0)
                fire(lax.rem(i0 + 2, NCHUNK), 0)
                drain(1)
                compute(i0 + 1, 1)
                fire(lax.rem(i0 + 3, NCHUNK), 1)
                return 0
            lax.fori_loop(0, NCHUNK // 2, _pair, 0)
            drain(0)
            drain(1)

        def _fire_half(h, with_b):
            def fire(i, st):
                base = _chunk_base(i)
                pltpu.sync_copy(srcf.at[pl.ds(base, C)], sidx[st])
                pltpu.async_copy(a_h[h].at[sidx[st]], bufa[st], sem)
                if with_b:
                    pltpu.sync_copy(dstf.at[pl.ds(base, C)], didx[st])
                    pltpu.async_copy(b_h[h].at[didx[st]], bufb[st], sem)
                    pltpu.async_copy(
                        e_h[h].at[pl.ds(base, C), :], bufe[st], sem)
                elif with_b is None:
                    pltpu.sync_copy(dstf.at[pl.ds(base, C)], didx[st])
            return fire

        def _drain_half(h, with_b):
            def drain(st):
                pltpu.make_async_copy(
                    a_h[h].at[sidx[st]], bufa[st], sem).wait()
                if with_b:
                    pltpu.make_async_copy(
                        b_h[h].at[didx[st]], bufb[st], sem).wait()
                    pltpu.make_async_copy(
                        e_h[h].at[pl.ds(0, C), :], bufe[st], sem).wait()
            return drain

        def _fill_didx2(st):
            def f(t, _):
                didx2[0, pl.ds(t * L, L)] = didx[st][pl.ds(t * L, L)]
                return 0
            lax.fori_loop(0, C // L, f, 0)

        lane16 = lane * L

        def _colsum():
            tot16 = plsc.load_gather(tbuf, [lane16])
            for k in range(1, L):
                tot16 = tot16 + plsc.load_gather(tbuf, [lane16 + k])
            return tot16

        def _partial_alpha(st, r, h):
            acc16 = zero16
            for c8 in range(8):
                sl = pl.ds(c8 * L, L)
                sv = bufa[st][r, sl] + bufb[st][r, sl] + bufe[st][r, sl]
                acc16 = acc16 + attv[h][c8] * jnp.maximum(sv, 0.2 * sv)
            return acc16

        if NH == 2:
            def _compute_a1(i, st):
                def _group(g, _):
                    def _e4(q, _2):
                        for j4 in range(4):
                            j = q * 4 + j4
                            r = g * L + j
                            tbuf[pl.ds(j * L, L)] = _partial_alpha(st, r, 1)
                        return 0
                    lax.fori_loop(0, 4, _e4, 0)
                    exloc[pl.ds(i * C + g * L, L)] = _colsum()
                    return 0
                lax.fori_loop(0, C // L, _group, 0)

            _pipelined(_fire_half(1, True), _drain_half(1, True),
                       _compute_a1)

        def _compute_a2(i, st):
            _fill_didx2(st)

            def _group(g, _):
                def _e4(q, _2):
                    for j4 in range(4):
                        j = q * 4 + j4
                        r = g * L + j
                        tbuf[pl.ds(j * L, L)] = _partial_alpha(st, r, 0)
                    return 0
                lax.fori_loop(0, 4, _e4, 0)
                tot16 = _colsum()
                if NH == 2:
                    tot16 = tot16 + exloc[pl.ds(i * C + g * L, L)]
                ex16 = jnp.exp(tot16)

                def _s4(q, _2):
                    for j4 in range(4):
                        j = q * 4 + j4
                        r = g * L + j
                        e_b = _lane_bcast(ex16, j)
                        for c8 in range(8):
                            sl = pl.ds(c8 * L, L)
                            bufa[st][r, sl] = bufa[st][r, sl] * e_b
                    return 0
                lax.fori_loop(0, 4, _s4, 0)

                dst16 = didx[st][pl.ds(g * L, L)]
                plsc.addupdate_scatter(denloc, [dst16], ex16)
                exloc[pl.ds(i * C + g * L, L)] = ex16
                return 0

            lax.fori_loop(0, C // L, _group, 0)
            pltpu.sync_copy(bufa[st], acc.at[didx2.at[0]], add=True)

        _pipelined(_fire_half(0, True), _drain_half(0, True), _compute_a2)

        pltpu.sync_copy(denloc, den_hbm.at[w])
        pltpu.sync_copy(exloc, ex_hbm.at[pl.ds(ebase, EPT)])
        plsc.subcore_barrier()
        pltpu.sync_copy(acc.at[pl.ds(row0, ROWS_PER_TILE)],
                        out_hbm.at[c, 0, pl.ds(row0, ROWS_PER_TILE)])

        if NH == 2:
            _zero_acc()
            plsc.subcore_barrier()

            def _compute_d(i, st):
                _fill_didx2(st)

                def _group(g, _):
                    exg = exloc[pl.ds(i * C + g * L, L)]

                    def _e4(q, _2):
                        for j4 in range(4):
                            j = q * 4 + j4
                            r = g * L + j
                            e_b = _lane_bcast(exg, j)
                            for c8 in range(8):
                                sl = pl.ds(c8 * L, L)
                                bufa[st][r, sl] = bufa[st][r, sl] * e_b
                        return 0
                    lax.fori_loop(0, 4, _e4, 0)
                    return 0

                lax.fori_loop(0, C // L, _group, 0)
                pltpu.sync_copy(bufa[st], acc.at[didx2.at[0]], add=True)

            _pipelined(_fire_half(1, None), _drain_half(1, False),
                       _compute_d)
            plsc.subcore_barrier()
            pltpu.sync_copy(acc.at[pl.ds(row0, ROWS_PER_TILE)],
                            out_hbm.at[c, 1, pl.ds(row0, ROWS_PER_TILE)])
    return body


def _gat_edge_phase(srcf, dstf, a_halves, b_halves, e_halves, att):
    NH = len(a_halves)
    k = pl.kernel(
        _make_layer_body(NH),
        out_type=[
            jax.ShapeDtypeStruct((NC, NH, N_PAD, 128), jnp.float32),
            jax.ShapeDtypeStruct((NW, N_PAD), jnp.float32),
            jax.ShapeDtypeStruct((E_FP,), jnp.float32),
        ],
        mesh=_mesh(),
        scratch_types=(
            [pltpu.VMEM_SHARED((N_PAD, 128), jnp.float32)]
            + [pltpu.VMEM((C, 128), jnp.float32)] * 6
            + [
                pltpu.VMEM((EPT,), jnp.float32),
                pltpu.VMEM((N_PAD,), jnp.float32),
                pltpu.VMEM((NH * 128,), jnp.float32),
                pltpu.VMEM((L * L,), jnp.float32),
                pltpu.VMEM((C,), jnp.int32),
                pltpu.VMEM((C,), jnp.int32),
                pltpu.VMEM((C,), jnp.int32),
                pltpu.VMEM((C,), jnp.int32),
                pltpu.VMEM((1, C), jnp.int32),
                pltpu.SemaphoreType.DMA,
            ]),
        **_SC_PARAMS,
    )
    return k(srcf, dstf, *a_halves, *b_halves, *e_halves, att)


C3 = 1296


def _attw_body(ex_hbm, dst_hbm, den_hbm, aw_hbm, dbuf, tmp, exbuf, dstbuf,
               awbuf, sem):
    c = lax.axis_index("c")
    s = lax.axis_index("s")
    w = c * NS + s
    ebase = w * EPT

    pltpu.sync_copy(den_hbm.at[0], dbuf)

    def _accum(t, _):
        pltpu.sync_copy(den_hbm.at[t], tmp)

        def _add(i, _2):
            sl = pl.ds(i * L, L)
            dbuf[sl] = dbuf[sl] + tmp[sl]
            return 0
        lax.fori_loop(0, N_PAD // L, _add, 0)
        return 0
    lax.fori_loop(1, NW, _accum, 0)

    def _chunk(i, _):
        base = pl.multiple_of(ebase + i * C3, 8)
        pltpu.sync_copy(ex_hbm.at[pl.ds(base, C3)], exbuf)
        pltpu.sync_copy(dst_hbm.at[pl.ds(base, C3)], dstbuf)

        def _group(g, _2):
            sl = pl.ds(g * L, L)
            d16 = plsc.load_gather(dbuf, [dstbuf[sl]])
            awbuf[sl] = exbuf[sl] / d16
            return 0
        lax.fori_loop(0, C3 // L, _group, 0)
        pltpu.sync_copy(awbuf, aw_hbm.at[pl.ds(base, C3)])
        return 0
    lax.fori_loop(0, EPT // C3, _chunk, 0)


def _attw(ex, dstf, den):
    k = pl.kernel(
        _attw_body,
        out_type=jax.ShapeDtypeStruct((E_FP,), jnp.float32),
        mesh=_mesh(),
        scratch_types=[
            pltpu.VMEM((N_PAD,), jnp.float32),
            pltpu.VMEM((N_PAD,), jnp.float32),
            pltpu.VMEM((C3,), jnp.float32),
            pltpu.VMEM((C3,), jnp.int32),
            pltpu.VMEM((C3,), jnp.float32),
            pltpu.SemaphoreType.DMA,
        ],
        **_SC_PARAMS,
    )
    return k(ex, dstf, den)


EBLK = 2048


def _e_proj(ea8, w8s):
    n_out = len(w8s)
    blk8 = EBLK // 8

    def body(x_ref, *refs):
        w_refs = refs[:n_out]
        o_refs = refs[n_out:]
        x = x_ref[...]
        for k in range(n_out):
            o_refs[k][...] = jnp.dot(
                x, w_refs[k][...], preferred_element_type=jnp.float32)

    grid = (E_FP // EBLK,)
    return pl.pallas_call(
        body,
        grid=grid,
        in_specs=[pl.BlockSpec((blk8, 128), lambda i: (i, 0))]
        + [pl.BlockSpec((128, 1024), lambda i: (0, 0))] * n_out,
        out_specs=[pl.BlockSpec((blk8, 1024), lambda i: (i, 0))] * n_out,
        out_shape=[jax.ShapeDtypeStruct((E_FP // 8, 1024), jnp.float32)] * n_out,
    )(ea8, *w8s)


def _x_proj(xp, ws):
    n_out = len(ws)
    K = xp.shape[1]

    def body(x_ref, *refs):
        w_refs = refs[:n_out]
        o_refs = refs[n_out:]
        x = x_ref[...]
        for k in range(n_out):
            o_refs[k][...] = jnp.dot(
                x, w_refs[k][...], preferred_element_type=jnp.float32)

    grid = (N_PAD // 256,)
    return pl.pallas_call(
        body,
        grid=grid,
        in_specs=[pl.BlockSpec((256, K), lambda i: (i, 0))]
        + [pl.BlockSpec((K, 128), lambda i: (0, 0))] * n_out,
        out_specs=[pl.BlockSpec((256, 128), lambda i: (i, 0))] * n_out,
        out_shape=[jax.ShapeDtypeStruct((N_PAD, 128), jnp.float32)] * n_out,
    )(xp, *ws)


def _finalize(out_p, den_p, bias_t, ws, relu):
    NH = out_p.shape[1]
    dout = NH * 128
    n_out = len(ws)

    def body(o_ref, d_ref, b_ref, *refs):
        w_refs = refs[:n_out]
        res = refs[n_out:]
        h_ref = res[0]
        o_res = res[1:]
        den = jnp.sum(d_ref[...], axis=0)
        halves = [o_ref[0, h] + o_ref[1, h] for h in range(NH)]
        hcat = jnp.concatenate(halves, axis=1) if NH > 1 else halves[0]
        fin = hcat / den[:, None] + b_ref[0:1, :]
        if relu:
            fin = jnp.maximum(fin, 0.0)
        row = (pl.program_id(0) * 256
               + lax.broadcasted_iota(jnp.int32, (256, 1), 0))
        fin = jnp.where(row < N, fin, 0.0)
        h_ref[...] = fin
        for k in range(n_out):
            o_res[k][...] = jnp.dot(
                fin, w_refs[k][...], preferred_element_type=jnp.float32)

    grid = (N_PAD // 256,)
    return pl.pallas_call(
        body,
        grid=grid,
        in_specs=[
            pl.BlockSpec((NC, NH, 256, 128), lambda i: (0, 0, i, 0)),
            pl.BlockSpec((NW, 256), lambda i: (0, i)),
            pl.BlockSpec((8, dout), lambda i: (0, 0)),
        ] + [pl.BlockSpec((dout, 128), lambda i: (0, 0))] * n_out,
        out_specs=[pl.BlockSpec((256, dout), lambda i: (i, 0))]
        + [pl.BlockSpec((256, 128), lambda i: (i, 0))] * n_out,
        out_shape=[jax.ShapeDtypeStruct((N_PAD, dout), jnp.float32)]
        + [jax.ShapeDtypeStruct((N_PAD, 128), jnp.float32)] * n_out,
    )(out_p, den_p, bias_t, *ws)


def _pool(h3, batchf):
    def body(h_ref, b_ref, emb_ref, gs_ref, gc_ref):
        i = pl.program_id(0)

        @pl.when(i == 0)
        def _init():
            gs_ref[...] = jnp.zeros_like(gs_ref)
            gc_ref[...] = jnp.zeros_like(gc_ref)

        b_ = b_ref[...]
        gid = lax.broadcasted_iota(jnp.int32, (256, G), 1)
        oh = (b_[:, None] == gid).astype(jnp.float32)
        gs_ref[...] += jnp.dot(oh.T, h_ref[...],
                               preferred_element_type=jnp.float32)
        gc_ref[...] += jnp.sum(oh, axis=0, keepdims=True)

        @pl.when(i == N_PAD // 256 - 1)
        def _fin():
            cnt = jnp.maximum(gc_ref[0, :], 1.0)
            emb_ref[...] = gs_ref[...] / cnt[:, None]

    grid = (N_PAD // 256,)
    return pl.pallas_call(
        body,
        grid=grid,
        in_specs=[
            pl.BlockSpec((256, 256), lambda i: (i, 0)),
            pl.BlockSpec((256,), lambda i: (i,)),
        ],
        out_specs=pl.BlockSpec((G, 256), lambda i: (0, 0)),
        out_shape=jax.ShapeDtypeStruct((G, 256), jnp.float32),
        scratch_shapes=[
            pltpu.VMEM((G, 256), jnp.float32),
            pltpu.VMEM((1, G), jnp.float32),
        ],
    )(h3, batchf)


def _kron8(w):
    return jnp.kron(jnp.eye(8, dtype=w.dtype), w)


def kernel(x, edge_index, edge_attr, batch, Wl1, Wr1, We1, att1, b1, Wl2, Wr2, We2, att2, b2, Wl3, Wr3, We3, att3, b3):
    src, dst = edge_index[0], edge_index[1]
    loop = jnp.arange(N, dtype=src.dtype)
    src_f = jnp.concatenate([src, loop])
    dst_f = jnp.concatenate([dst, loop])
    ei_ret = jnp.stack([src_f, dst_f])

    pad = E_AP - E
    ea_pad = jnp.concatenate(
        [edge_attr, jnp.zeros((pad, ED), jnp.float32)], axis=0).reshape(-1)
    dst_pad = jnp.concatenate(
        [dst.astype(jnp.int32), jnp.full((pad,), DUMMY, jnp.int32)])
    stats = _edge_stats(ea_pad, dst_pad)
    stats = stats[0] + stats[1]
    sums = stats[:N, :ED]
    cnt = stats[:N, ED]
    loop_attr = sums / jnp.clip(cnt, 1.0)[:, None]

    epad = E_FP - E_F
    srcf_p = jnp.concatenate(
        [src_f.astype(jnp.int32), jnp.zeros((epad,), jnp.int32)])
    dstf_p = jnp.concatenate(
        [dst_f.astype(jnp.int32), jnp.full((epad,), DUMMY, jnp.int32)])
    eaf_p = jnp.concatenate(
        [edge_attr, loop_attr, jnp.zeros((epad, ED), jnp.float32)], axis=0)
    ea8 = eaf_p.reshape(E_FP // 8, 8 * ED)

    w8s = [_kron8(We1), _kron8(We2[:, :128]), _kron8(We2[:, 128:]),
           _kron8(We3[:, :128]), _kron8(We3[:, 128:])]
    e1, e2L, e2H, e3L, e3H = [
        o.reshape(E_FP, 128) for o in _e_proj(ea8, w8s)]

    xp = jnp.concatenate(
        [x, jnp.zeros((N_PAD - N, x.shape[1]), jnp.float32)], axis=0)
    a1, bm1 = _x_proj(xp, [Wl1, Wr1])

    out1, den1, _ = _gat_edge_phase(srcf_p, dstf_p, [a1], [bm1], [e1], att1)
    b1t = jnp.tile(b1[None, :], (8, 1))
    _, a2L, a2H, b2L, b2H = _finalize(
        out1, den1, b1t, [Wl2[:, :128], Wl2[:, 128:],
                          Wr2[:, :128], Wr2[:, 128:]], relu=True)

    out2, den2, _ = _gat_edge_phase(
        srcf_p, dstf_p, [a2L, a2H], [b2L, b2H], [e2L, e2H], att2)
    b2t = jnp.tile(b2[None, :], (8, 1))
    _, a3L, a3H, b3L, b3H = _finalize(
        out2, den2, b2t, [Wl3[:, :128], Wl3[:, 128:],
                          Wr3[:, :128], Wr3[:, 128:]], relu=True)

    out3, den3, ex3 = _gat_edge_phase(
        srcf_p, dstf_p, [a3L, a3H], [b3L, b3H], [e3L, e3H], att3)
    b3t = jnp.tile(b3[None, :], (8, 1))
    (h3,) = _finalize(out3, den3, b3t, [], relu=False)

    batchf = jnp.concatenate(
        [batch.astype(jnp.int32), jnp.full((N_PAD - N,), G, jnp.int32)])
    emb = _pool(h3, batchf)

    aw = _attw(ex3, dstf_p, den3)
    att_w = aw[:E_F]

    return emb, ei_ret, att_w
